# baseline (speedup 1.0000x reference)
import jax
import jax.numpy as jnp
from jax import lax
from jax.experimental import pallas as pl
from jax.experimental.pallas import tpu as pltpu
from jax.experimental.pallas import tpu_sc as plsc

B = 16384
D = 32
NC = 2
NS = 16
L = 16
NW = NC * NS
BPW = B // NW
NCH = 4
CH = BPW // NCH
GCH = CH // L


def _body(uidx_hbm, iidx_hbm, table_hbm, out_hbm,
          uidx_v, iidx_v, urows_v, irows_v, out_v, sems):
    wid = lax.axis_index("s") * NC + lax.axis_index("c")
    base = wid * BPW

    pltpu.sync_copy(uidx_hbm.at[pl.ds(base, BPW)], uidx_v)
    pltpu.sync_copy(iidx_hbm.at[pl.ds(base, BPW)], iidx_v)

    def start(c):
        buf = c % 2
        cu = pltpu.async_copy(
            table_hbm.at[uidx_v.at[pl.ds(c * CH, CH)]],
            urows_v.at[buf], sems.at[buf, 0])
        ci = pltpu.async_copy(
            table_hbm.at[iidx_v.at[pl.ds(c * CH, CH)]],
            irows_v.at[buf], sems.at[buf, 1])
        return cu, ci

    lane = lax.iota(jnp.int32, L)
    inflight = start(0)

    for c in range(NCH):
        if c + 1 < NCH:
            nxt = start(c + 1)
        cu, ci = inflight
        cu.wait()
        ci.wait()
        buf = c % 2

        @plsc.parallel_loop(0, GCH)
        def group(g):
            rid = g * L + lane
            acc = jnp.zeros((L,), jnp.float32)
            for d in range(D):
                cid = jnp.full((L,), d, jnp.int32)
                uv = plsc.load_gather(urows_v.at[buf], [rid, cid])
                iv = plsc.load_gather(irows_v.at[buf], [rid, cid])
                acc = acc + uv * iv
            out_v[pl.ds(c * CH + g * L, L)] = acc

        if c + 1 < NCH:
            inflight = nxt

    pltpu.sync_copy(out_v, out_hbm.at[pl.ds(base, BPW)])


def kernel(edge_index, edge_label_index, embedding_weight):
    del edge_index
    uidx = edge_label_index[0]
    iidx = edge_label_index[1]
    mesh = plsc.VectorSubcoreMesh(core_axis_name="c", subcore_axis_name="s")
    f = pl.kernel(
        _body,
        out_type=jax.ShapeDtypeStruct((B,), jnp.float32),
        mesh=mesh,
        compiler_params=pltpu.CompilerParams(
            needs_layout_passes=False, use_tc_tiling_on_sc=False
        ),
        scratch_types=[
            pltpu.VMEM((BPW,), jnp.int32),
            pltpu.VMEM((BPW,), jnp.int32),
            pltpu.VMEM((2, CH, D), jnp.float32),
            pltpu.VMEM((2, CH, D), jnp.float32),
            pltpu.VMEM((BPW,), jnp.float32),
            pltpu.SemaphoreType.DMA((2, 2)),
        ],
    )
    return f(uidx, iidx, embedding_weight)
